# baseline (speedup 1.0000x reference)
import functools

import jax
import jax.numpy as jnp
from jax import lax
from jax.experimental import pallas as pl
from jax.experimental.pallas import tpu as pltpu
from jax.experimental.pallas import tpu_sc as plsc

_NC = 2
_NS = 16
_NW = _NC * _NS
_L = 16


@functools.lru_cache(maxsize=None)
def _build(B, D, K):
    N = B * D
    assert N % (_NW * _L) == 0
    chunk = N // _NW
    steps = chunk // _L

    mesh = plsc.VectorSubcoreMesh(
        core_axis_name="c", subcore_axis_name="s",
        num_cores=_NC, num_subcores=_NS,
    )

    @functools.partial(
        pl.kernel,
        out_type=(
            jax.ShapeDtypeStruct((N,), jnp.float32),
            jax.ShapeDtypeStruct((N,), jnp.float32),
        ),
        mesh=mesh,
        scratch_types=[
            pltpu.VMEM((chunk,), jnp.int32),
            pltpu.VMEM((chunk,), jnp.int32),
            pltpu.VMEM((chunk,), jnp.float32),
            pltpu.VMEM((chunk,), jnp.float32),
            pltpu.SemaphoreType.DMA,
            pltpu.SemaphoreType.DMA,
        ],
    )
    def gather_kernel(labels_hbm, mean_hbm, lv_hbm, out_m_hbm, out_v_hbm,
                      lab_v, idx_v, m_v, v_v, sem_m, sem_v):
        wid = lax.axis_index("s") * _NC + lax.axis_index("c")
        base = wid * chunk
        pltpu.sync_copy(labels_hbm.at[pl.ds(base, chunk)], lab_v)

        lane = lax.iota(jnp.int32, _L)

        def step(j, carry):
            off = j * _L
            lab = lab_v[pl.ds(off, _L)]
            pos = base + off + lane
            d = lax.rem(pos, D)
            idx_v[pl.ds(off, _L)] = lab + d * K
            return carry

        lax.fori_loop(0, steps, step, 0)

        cp_m = pltpu.async_copy(mean_hbm.at[idx_v], m_v, sem_m)
        cp_v = pltpu.async_copy(lv_hbm.at[idx_v], v_v, sem_v)
        cp_m.wait()
        cp_v.wait()
        pltpu.sync_copy(m_v, out_m_hbm.at[pl.ds(base, chunk)])
        pltpu.sync_copy(v_v, out_v_hbm.at[pl.ds(base, chunk)])

    return gather_kernel


def kernel(labels, mean, log_var):
    B, D = labels.shape
    K = mean.shape[1]
    gk = _build(B, D, K)
    out_m, out_v = gk(
        labels.astype(jnp.int32).reshape(-1),
        mean.reshape(-1),
        log_var.reshape(-1),
    )
    return out_m.reshape(B, D), out_v.reshape(B, D)


# speedup vs baseline: 1.6368x; 1.6368x over previous
import functools

import jax
import jax.numpy as jnp
from jax import lax
from jax.experimental import pallas as pl
from jax.experimental.pallas import tpu as pltpu
from jax.experimental.pallas import tpu_sc as plsc

_NC = 2
_NS = 16
_NW = _NC * _NS
_L = 16
_NB = 4


@functools.lru_cache(maxsize=None)
def _build(B, D, K):
    N = B * D
    assert B == 1 << 14, "index math below uses B = 2**14"
    assert N % (_NW * _NB * _L) == 0
    chunk = N // _NW
    sb = chunk // _NB
    steps = sb // _L

    mesh = plsc.VectorSubcoreMesh(
        core_axis_name="c", subcore_axis_name="s",
        num_cores=_NC, num_subcores=_NS,
    )

    @functools.partial(
        pl.kernel,
        out_type=jax.ShapeDtypeStruct((N,), jnp.float32),
        mesh=mesh,
        scratch_types=[
            pltpu.VMEM((chunk,), jnp.int32),
            [pltpu.VMEM((sb,), jnp.int32) for _ in range(_NB)],
            [pltpu.VMEM((sb,), jnp.float32) for _ in range(_NB)],
            [pltpu.SemaphoreType.DMA for _ in range(_NB)],
            pltpu.SemaphoreType.DMA,
        ],
    )
    def gather_kernel(lab_hbm, tab_hbm, out_hbm, lab_v, idx_vs, g_vs,
                      sems_g, sem_w):
        w = lax.axis_index("s") * _NC + lax.axis_index("c")
        base = w * chunk
        pltpu.sync_copy(lab_hbm.at[pl.ds(base, chunk)], lab_v)

        lane = lax.iota(jnp.int32, _L)

        cps = []
        for blk in range(_NB):
            idx_v = idx_vs[blk]
            bbase = base + blk * sb

            def step(j, carry, idx_v=idx_v, bbase=bbase, boff=blk * sb):
                off = j * _L
                lab = lab_v[pl.ds(boff + off, _L)]
                d = lax.shift_right_logical(bbase + off + lane, 14)
                idx_v[pl.ds(off, _L)] = lab + d * K
                return carry

            lax.fori_loop(0, steps, step, 0)
            cps.append(pltpu.async_copy(
                tab_hbm.at[idx_vs[blk]], g_vs[blk], sems_g[blk]))

        wbs = []
        for blk in range(_NB):
            cps[blk].wait()
            wbs.append(pltpu.async_copy(
                g_vs[blk], out_hbm.at[pl.ds(base + blk * sb, sb)], sem_w))
        for wb in wbs:
            wb.wait()

    return gather_kernel


def kernel(labels, mean, log_var):
    B, D = labels.shape
    K = mean.shape[1]
    gk = _build(B, D, K)
    lab_flat = jnp.transpose(labels.astype(jnp.int32)).reshape(-1)
    outm_f = gk(lab_flat, mean.reshape(-1))
    outv_f = gk(lab_flat, log_var.reshape(-1))
    return (
        jnp.transpose(outm_f.reshape(D, B)),
        jnp.transpose(outv_f.reshape(D, B)),
    )


# speedup vs baseline: 1.6396x; 1.0017x over previous
import functools

import jax
import jax.numpy as jnp
from jax import lax
from jax.experimental import pallas as pl
from jax.experimental.pallas import tpu as pltpu
from jax.experimental.pallas import tpu_sc as plsc

_NC = 2
_NS = 16
_NW = _NC * _NS
_L = 16
_NB = 8


@functools.lru_cache(maxsize=None)
def _build(B, D, K):
    N = B * D
    assert B == 1 << 14, "index math below uses B = 2**14"
    assert N % (_NW * _NB * _L) == 0
    chunk = N // _NW
    sb = chunk // _NB
    steps = sb // _L

    mesh = plsc.VectorSubcoreMesh(
        core_axis_name="c", subcore_axis_name="s",
        num_cores=_NC, num_subcores=_NS,
    )

    @functools.partial(
        pl.kernel,
        out_type=jax.ShapeDtypeStruct((N,), jnp.float32),
        mesh=mesh,
        scratch_types=[
            pltpu.VMEM((chunk,), jnp.int32),
            [pltpu.VMEM((sb,), jnp.int32) for _ in range(_NB)],
            [pltpu.VMEM((sb,), jnp.float32) for _ in range(_NB)],
            [pltpu.SemaphoreType.DMA for _ in range(_NB)],
            pltpu.SemaphoreType.DMA,
        ],
    )
    def gather_kernel(lab_hbm, tab_hbm, out_hbm, lab_v, idx_vs, g_vs,
                      sems_g, sem_w):
        w = lax.axis_index("s") * _NC + lax.axis_index("c")
        base = w * chunk
        pltpu.sync_copy(lab_hbm.at[pl.ds(base, chunk)], lab_v)

        lane = lax.iota(jnp.int32, _L)

        cps = []
        for blk in range(_NB):
            idx_v = idx_vs[blk]
            bbase = base + blk * sb

            def step(j, carry, idx_v=idx_v, bbase=bbase, boff=blk * sb):
                off = j * _L
                lab = lab_v[pl.ds(boff + off, _L)]
                d = lax.shift_right_logical(bbase + off + lane, 14)
                idx_v[pl.ds(off, _L)] = lab + d * K
                return carry

            lax.fori_loop(0, steps, step, 0)
            cps.append(pltpu.async_copy(
                tab_hbm.at[idx_vs[blk]], g_vs[blk], sems_g[blk]))

        wbs = []
        for blk in range(_NB):
            cps[blk].wait()
            wbs.append(pltpu.async_copy(
                g_vs[blk], out_hbm.at[pl.ds(base + blk * sb, sb)], sem_w))
        for wb in wbs:
            wb.wait()

    return gather_kernel


def kernel(labels, mean, log_var):
    B, D = labels.shape
    K = mean.shape[1]
    gk = _build(B, D, K)
    lab_flat = jnp.transpose(labels.astype(jnp.int32)).reshape(-1)
    outm_f = gk(lab_flat, mean.reshape(-1))
    outv_f = gk(lab_flat, log_var.reshape(-1))
    return (
        jnp.transpose(outm_f.reshape(D, B)),
        jnp.transpose(outv_f.reshape(D, B)),
    )


# speedup vs baseline: 2.7177x; 1.6576x over previous
import functools

import jax
import jax.numpy as jnp
from jax import lax
from jax.experimental import pallas as pl
from jax.experimental.pallas import tpu as pltpu
from jax.experimental.pallas import tpu_sc as plsc

_NC = 2
_NS = 16


@functools.lru_cache(maxsize=None)
def _build(B, D, K):
    N = B * D
    assert D % _NC == 0
    assert B % _NS == 0
    dpc = D // _NC
    bpt = B // _NS

    mesh = plsc.VectorSubcoreMesh(
        core_axis_name="c", subcore_axis_name="s",
        num_cores=_NC, num_subcores=_NS,
    )

    @functools.partial(
        pl.kernel,
        out_type=(
            jax.ShapeDtypeStruct((N,), jnp.float32),
            jax.ShapeDtypeStruct((N,), jnp.float32),
        ),
        mesh=mesh,
        scratch_types=[
            [pltpu.VMEM_SHARED((K,), jnp.float32) for _ in range(2)],
            [pltpu.VMEM_SHARED((K,), jnp.float32) for _ in range(2)],
            [pltpu.VMEM((bpt,), jnp.int32) for _ in range(dpc)],
            [pltpu.VMEM((bpt,), jnp.float32) for _ in range(2)],
            [pltpu.VMEM((bpt,), jnp.float32) for _ in range(2)],
            pltpu.SemaphoreType.DMA,
            [pltpu.SemaphoreType.DMA for _ in range(2)],
            [pltpu.SemaphoreType.DMA for _ in range(2)],
            [pltpu.SemaphoreType.DMA for _ in range(2)],
            [pltpu.SemaphoreType.DMA for _ in range(2)],
            [pltpu.SemaphoreType.DMA for _ in range(2)],
            [pltpu.SemaphoreType.DMA for _ in range(2)],
        ],
    )
    def gather_kernel(lab_hbm, mean_hbm, lv_hbm, outm_hbm, outv_hbm,
                      sm_slots, sv_slots, lab_vs, gm_vs, gv_vs,
                      sem_lab, sems_sm, sems_sv, sems_gm, sems_gv,
                      sems_wm, sems_wv):
        c = lax.axis_index("c")
        s = lax.axis_index("s")
        d0 = c * dpc

        lab_cps = []
        for dd in range(dpc):
            lab_cps.append(pltpu.async_copy(
                lab_hbm.at[pl.ds((d0 + dd) * B + s * bpt, bpt)],
                lab_vs[dd], sem_lab))

        def stage(dd):
            sl = dd % 2

            @pl.when(s == (2 * dd) % _NS)
            def _():
                pltpu.async_copy(
                    mean_hbm.at[d0 + dd], sm_slots[sl], sems_sm[sl])

            @pl.when(s == (2 * dd + 1) % _NS)
            def _():
                pltpu.async_copy(
                    lv_hbm.at[d0 + dd], sv_slots[sl], sems_sv[sl])

        def stage_wait(dd):
            sl = dd % 2

            @pl.when(s == (2 * dd) % _NS)
            def _():
                pltpu.make_async_copy(
                    mean_hbm.at[d0 + dd], sm_slots[sl], sems_sm[sl]).wait()

            @pl.when(s == (2 * dd + 1) % _NS)
            def _():
                pltpu.make_async_copy(
                    lv_hbm.at[d0 + dd], sv_slots[sl], sems_sv[sl]).wait()

        stage(0)
        wbs = {}
        for dd in range(dpc):
            sl = dd % 2
            stage_wait(dd)
            plsc.subcore_barrier()
            if dd + 1 < dpc:
                stage(dd + 1)
            lab_cps[dd].wait()
            if dd >= 2:
                wm, wv = wbs.pop(dd - 2)
                wm.wait()
                wv.wait()
            cp_m = pltpu.async_copy(
                sm_slots[sl].at[lab_vs[dd]], gm_vs[sl], sems_gm[sl])
            cp_v = pltpu.async_copy(
                sv_slots[sl].at[lab_vs[dd]], gv_vs[sl], sems_gv[sl])
            cp_m.wait()
            cp_v.wait()
            out_slice = pl.ds((d0 + dd) * B + s * bpt, bpt)
            wm = pltpu.async_copy(gm_vs[sl], outm_hbm.at[out_slice],
                                  sems_wm[sl])
            wv = pltpu.async_copy(gv_vs[sl], outv_hbm.at[out_slice],
                                  sems_wv[sl])
            wbs[dd] = (wm, wv)
            plsc.subcore_barrier()

        for dd in sorted(wbs):
            wm, wv = wbs[dd]
            wm.wait()
            wv.wait()

    return gather_kernel


def kernel(labels, mean, log_var):
    B, D = labels.shape
    K = mean.shape[1]
    gk = _build(B, D, K)
    lab_flat = jnp.transpose(labels.astype(jnp.int32)).reshape(-1)
    outm_f, outv_f = gk(lab_flat, mean, log_var)
    return (
        jnp.transpose(outm_f.reshape(D, B)),
        jnp.transpose(outv_f.reshape(D, B)),
    )


# speedup vs baseline: 3.7262x; 1.3710x over previous
import functools

import jax
import jax.numpy as jnp
from jax import lax
from jax.experimental import pallas as pl
from jax.experimental.pallas import tpu as pltpu
from jax.experimental.pallas import tpu_sc as plsc

_NC = 2
_NS = 16
_SL = 4


@functools.lru_cache(maxsize=None)
def _build(B, D, K):
    assert D % _NC == 0
    assert B % _NS == 0
    dpc = D // _NC
    bpt = B // _NS

    mesh = plsc.VectorSubcoreMesh(
        core_axis_name="c", subcore_axis_name="s",
        num_cores=_NC, num_subcores=_NS,
    )

    @functools.partial(
        pl.kernel,
        out_type=(
            jax.ShapeDtypeStruct((D, B), jnp.float32),
            jax.ShapeDtypeStruct((D, B), jnp.float32),
        ),
        mesh=mesh,
        scratch_types=[
            [pltpu.VMEM_SHARED((K,), jnp.float32) for _ in range(_SL)],
            [pltpu.VMEM_SHARED((K,), jnp.float32) for _ in range(_SL)],
            [pltpu.VMEM((bpt,), jnp.int32) for _ in range(dpc)],
            [pltpu.VMEM((bpt,), jnp.float32) for _ in range(2)],
            [pltpu.VMEM((bpt,), jnp.float32) for _ in range(2)],
            pltpu.SemaphoreType.DMA,
            [pltpu.SemaphoreType.DMA for _ in range(_SL)],
            [pltpu.SemaphoreType.DMA for _ in range(_SL)],
            [pltpu.SemaphoreType.DMA for _ in range(2)],
            [pltpu.SemaphoreType.DMA for _ in range(2)],
            [pltpu.SemaphoreType.DMA for _ in range(2)],
            [pltpu.SemaphoreType.DMA for _ in range(2)],
        ],
    )
    def gather_kernel(labt_hbm, mean_hbm, lv_hbm, outm_hbm, outv_hbm,
                      sm_slots, sv_slots, lab_vs, gm_vs, gv_vs,
                      sem_lab, sems_sm, sems_sv, sems_gm, sems_gv,
                      sems_wm, sems_wv):
        c = lax.axis_index("c")
        s = lax.axis_index("s")
        d0 = c * dpc

        lab_cps = []
        for dd in range(dpc):
            lab_cps.append(pltpu.async_copy(
                labt_hbm.at[d0 + dd, pl.ds(s * bpt, bpt)],
                lab_vs[dd], sem_lab))

        def stage(dd):
            sl = dd % _SL

            @pl.when(s == (2 * dd) % _NS)
            def _():
                pltpu.async_copy(
                    mean_hbm.at[d0 + dd], sm_slots[sl], sems_sm[sl])

            @pl.when(s == (2 * dd + 1) % _NS)
            def _():
                pltpu.async_copy(
                    lv_hbm.at[d0 + dd], sv_slots[sl], sems_sv[sl])

        def stage_wait(dd):
            sl = dd % _SL

            @pl.when(s == (2 * dd) % _NS)
            def _():
                pltpu.make_async_copy(
                    mean_hbm.at[d0 + dd], sm_slots[sl], sems_sm[sl]).wait()

            @pl.when(s == (2 * dd + 1) % _NS)
            def _():
                pltpu.make_async_copy(
                    lv_hbm.at[d0 + dd], sv_slots[sl], sems_sv[sl]).wait()

        for dd in range(min(_SL - 1, dpc)):
            stage(dd)

        for cp in lab_cps:
            cp.wait()

        wbs = {}
        for dd in range(dpc):
            sl = dd % _SL
            pr = dd % 2
            stage_wait(dd)
            if dd >= 2:
                wm, wv = wbs.pop(dd - 2)
                wm.wait()
                wv.wait()
            plsc.subcore_barrier()
            if dd + _SL - 1 < dpc:
                stage(dd + _SL - 1)
            cp_m = pltpu.async_copy(
                sm_slots[sl].at[lab_vs[dd]], gm_vs[pr], sems_gm[pr])
            cp_v = pltpu.async_copy(
                sv_slots[sl].at[lab_vs[dd]], gv_vs[pr], sems_gv[pr])
            cp_m.wait()
            cp_v.wait()
            out_slice = pl.ds(s * bpt, bpt)
            wm = pltpu.async_copy(
                gm_vs[pr], outm_hbm.at[d0 + dd, out_slice], sems_wm[pr])
            wv = pltpu.async_copy(
                gv_vs[pr], outv_hbm.at[d0 + dd, out_slice], sems_wv[pr])
            wbs[dd] = (wm, wv)

        for dd in sorted(wbs):
            wm, wv = wbs[dd]
            wm.wait()
            wv.wait()

    return gather_kernel


def kernel(labels, mean, log_var):
    B, D = labels.shape
    K = mean.shape[1]
    gk = _build(B, D, K)
    labt = jnp.transpose(labels.astype(jnp.int32))
    outm_t, outv_t = gk(labt, mean, log_var)
    return jnp.transpose(outm_t), jnp.transpose(outv_t)
